# baseline (speedup 1.0000x reference)
import functools
import math

import jax
import jax.numpy as jnp
from jax import lax
from jax.experimental import pallas as pl
from jax.experimental.pallas import tpu as pltpu
from jax.experimental.pallas import tpu_sc as plsc

D_MODEL = 64
SCALE = math.sqrt(D_MODEL)
PADW = 128

NC = 2
NS = 16
NW = NC * NS

CHUNK = 512
NBUF = 3

VB = 8192
SB = 512
TB = 8


def _transpose_body(r_ref, o_ref):
    vt = r_ref[...].T
    o_ref[:, :D_MODEL] = vt[: VB // 2]
    o_ref[:, D_MODEL:] = vt[VB // 2 :]


def _gather_body(b_per_w, chunk, table_hbm, idx_hbm, raw_hbm, idx_v, bufs, gsem, wsem):
    wid = lax.axis_index("s") * NC + lax.axis_index("c")
    base = wid * b_per_w
    pltpu.sync_copy(idx_hbm.at[pl.ds(base, b_per_w)], idx_v)

    n = b_per_w // chunk

    def start_gather(g):
        b = g % NBUF
        return pltpu.async_copy(
            table_hbm.at[idx_v.at[pl.ds(g * chunk, chunk)]], bufs[b], gsem.at[b]
        )

    def start_write(g):
        b = g % NBUF
        dst = raw_hbm.at[pl.ds(base + g * chunk, chunk)]
        return pltpu.async_copy(bufs[b], dst, wsem.at[b])

    ghandles = [None] * n
    whandles = [None] * n
    for g in range(n):
        if g >= NBUF:
            whandles[g - NBUF].wait()
        ghandles[g] = start_gather(g)
        if g >= 1:
            ghandles[g - 1].wait()
            whandles[g - 1] = start_write(g - 1)
    ghandles[n - 1].wait()
    whandles[n - 1] = start_write(n - 1)
    for g in range(max(0, n - NBUF), n):
        whandles[g].wait()


def _out_body(r_ref, o_ref):
    r = r_ref[...]
    for t in range(TB):
        rt = r[:, t * D_MODEL : (t + 1) * D_MODEL]
        o_ref[t, :, :] = rt.T * SCALE


def _out_body_acc(prev_ref, r_ref, o_ref):
    del prev_ref
    r = r_ref[...]
    for t in range(TB):
        rt = r[:, t * D_MODEL : (t + 1) * D_MODEL]
        o_ref[t, :, :] = rt.T * SCALE


def kernel(x, table):
    S, T = x.shape
    B = S * T
    V = table.shape[0]
    b_per_w = B // NW
    assert b_per_w % CHUNK == 0

    n_vb = (V + VB - 1) // VB
    vpad = n_vb * VB
    table_ps = pl.pallas_call(
        _transpose_body,
        grid=(n_vb,),
        in_specs=[pl.BlockSpec((D_MODEL, VB), lambda i: (0, i))],
        out_specs=pl.BlockSpec((VB // 2, 2 * D_MODEL), lambda i: (i, 0)),
        out_shape=jax.ShapeDtypeStruct((vpad // 2, 2 * D_MODEL), jnp.float32),
    )(table.T)
    table_lin = table_ps.reshape(-1).reshape(vpad, D_MODEL)

    h = VB // 2
    xr = (x & ~(VB - 1)) + 2 * (x & (h - 1)) + ((x // h) & 1)

    NSL = 4
    s_sl = S // NSL
    b_sl = s_sl * T
    bw_sl = b_sl // NW
    chunk = 400
    assert bw_sl % chunk == 0

    mesh = plsc.VectorSubcoreMesh(
        core_axis_name="c", subcore_axis_name="s", num_cores=NC
    )
    gather = pl.kernel(
        functools.partial(_gather_body, bw_sl, chunk),
        out_type=jax.ShapeDtypeStruct((b_sl, D_MODEL), jnp.float32),
        mesh=mesh,
        compiler_params=pltpu.CompilerParams(use_tc_tiling_on_sc=False),
        scratch_types=[
            pltpu.VMEM((bw_sl,), jnp.int32),
            [pltpu.VMEM((chunk, D_MODEL), jnp.float32) for _ in range(NBUF)],
            pltpu.SemaphoreType.DMA((NBUF,)),
            pltpu.SemaphoreType.DMA((NBUF,)),
        ],
    )

    out_t = None
    for sl in range(NSL):
        xr_sl = xr[sl * s_sl : (sl + 1) * s_sl].reshape(-1)
        raw = gather(table_lin, xr_sl)
        raw2 = raw.reshape(s_sl, T * D_MODEL)
        base = sl * (s_sl // SB)
        if out_t is None:
            out_t = pl.pallas_call(
                _out_body,
                grid=(s_sl // SB, T // TB),
                in_specs=[
                    pl.BlockSpec((SB, TB * D_MODEL), lambda i, j: (i, j))
                ],
                out_specs=pl.BlockSpec(
                    (TB, D_MODEL, SB), lambda i, j, b=base: (j, 0, b + i)
                ),
                out_shape=jax.ShapeDtypeStruct((T, D_MODEL, S), jnp.float32),
            )(raw2)
        else:
            out_t = pl.pallas_call(
                _out_body_acc,
                grid=(s_sl // SB, T // TB),
                in_specs=[
                    pl.BlockSpec(memory_space=pltpu.MemorySpace.HBM),
                    pl.BlockSpec((SB, TB * D_MODEL), lambda i, j: (i, j)),
                ],
                out_specs=pl.BlockSpec(
                    (TB, D_MODEL, SB), lambda i, j, b=base: (j, 0, b + i)
                ),
                out_shape=jax.ShapeDtypeStruct((T, D_MODEL, S), jnp.float32),
                input_output_aliases={0: 0},
            )(out_t, raw2)
    return out_t.transpose(2, 0, 1)


# speedup vs baseline: 1.2094x; 1.2094x over previous
import functools
import math

import jax
import jax.numpy as jnp
from jax import lax
from jax.experimental import pallas as pl
from jax.experimental.pallas import tpu as pltpu
from jax.experimental.pallas import tpu_sc as plsc

D_MODEL = 64
SCALE = math.sqrt(D_MODEL)
PADW = 128

NC = 2
NS = 16
NW = NC * NS

CHUNK = 512
NBUF = 3

VB = 8192
SB = 512
TB = 8


def _transpose_body(r_ref, o_ref):
    vt = r_ref[...].T
    o_ref[:, :D_MODEL] = vt[: VB // 2]
    o_ref[:, D_MODEL:] = vt[VB // 2 :]


def _gather_body(b_per_w, chunk, table_hbm, idx_hbm, raw_hbm, idx_v, bufs, gsem, wsem):
    wid = lax.axis_index("s") * NC + lax.axis_index("c")
    base = wid * b_per_w
    pltpu.sync_copy(idx_hbm.at[pl.ds(base, b_per_w)], idx_v)

    n = b_per_w // chunk

    def start_gather(g):
        b = g % NBUF
        return pltpu.async_copy(
            table_hbm.at[idx_v.at[pl.ds(g * chunk, chunk)]], bufs[b], gsem.at[b]
        )

    def start_write(g):
        b = g % NBUF
        dst = raw_hbm.at[pl.ds(base + g * chunk, chunk), pl.ds(0, D_MODEL)]
        return pltpu.async_copy(bufs[b], dst, wsem.at[b])

    ghandles = [None] * n
    whandles = [None] * n
    for g in range(n):
        if g >= NBUF:
            whandles[g - NBUF].wait()
        ghandles[g] = start_gather(g)
        if g >= 1:
            ghandles[g - 1].wait()
            whandles[g - 1] = start_write(g - 1)
    ghandles[n - 1].wait()
    whandles[n - 1] = start_write(n - 1)
    for g in range(max(0, n - NBUF), n):
        whandles[g].wait()


def _out_body(r_ref, o_ref):
    r = r_ref[...]
    for t in range(TB):
        rt = r[:, t, :D_MODEL]
        o_ref[t, :, :] = rt.T * SCALE


def _out_body_acc(prev_ref, r_ref, o_ref):
    del prev_ref
    r = r_ref[...]
    for t in range(TB):
        rt = r[:, t, :D_MODEL]
        o_ref[t, :, :] = rt.T * SCALE


def kernel(x, table):
    S, T = x.shape
    B = S * T
    V = table.shape[0]
    b_per_w = B // NW
    assert b_per_w % CHUNK == 0

    n_vb = (V + VB - 1) // VB
    vpad = n_vb * VB
    table_ps = pl.pallas_call(
        _transpose_body,
        grid=(n_vb,),
        in_specs=[pl.BlockSpec((D_MODEL, VB), lambda i: (0, i))],
        out_specs=pl.BlockSpec((VB // 2, 2 * D_MODEL), lambda i: (i, 0)),
        out_shape=jax.ShapeDtypeStruct((vpad // 2, 2 * D_MODEL), jnp.float32),
    )(table.T)
    table_lin = table_ps.reshape(-1).reshape(vpad, D_MODEL)

    h = VB // 2
    xr = (x & ~(VB - 1)) + 2 * (x & (h - 1)) + ((x // h) & 1)

    NSL = 4
    s_sl = S // NSL
    b_sl = s_sl * T
    bw_sl = b_sl // NW
    chunk = 400
    assert bw_sl % chunk == 0

    mesh = plsc.VectorSubcoreMesh(
        core_axis_name="c", subcore_axis_name="s", num_cores=NC
    )
    gather = pl.kernel(
        functools.partial(_gather_body, bw_sl, chunk),
        out_type=jax.ShapeDtypeStruct((b_sl, PADW), jnp.float32),
        mesh=mesh,
        compiler_params=pltpu.CompilerParams(use_tc_tiling_on_sc=False),
        scratch_types=[
            pltpu.VMEM((bw_sl,), jnp.int32),
            [pltpu.VMEM((chunk, D_MODEL), jnp.float32) for _ in range(NBUF)],
            pltpu.SemaphoreType.DMA((NBUF,)),
            pltpu.SemaphoreType.DMA((NBUF,)),
        ],
    )

    out_t = None
    for sl in range(NSL):
        xr_sl = xr[sl * s_sl : (sl + 1) * s_sl].reshape(-1)
        raw = gather(table_lin, xr_sl)
        raw3 = raw.reshape(s_sl, T, PADW)
        base = sl * (s_sl // SB)
        if out_t is None:
            out_t = pl.pallas_call(
                _out_body,
                grid=(s_sl // SB, T // TB),
                in_specs=[
                    pl.BlockSpec((SB, TB, PADW), lambda i, j: (i, j, 0))
                ],
                out_specs=pl.BlockSpec(
                    (TB, D_MODEL, SB), lambda i, j, b=base: (j, 0, b + i)
                ),
                out_shape=jax.ShapeDtypeStruct((T, D_MODEL, S), jnp.float32),
            )(raw3)
        else:
            out_t = pl.pallas_call(
                _out_body_acc,
                grid=(s_sl // SB, T // TB),
                in_specs=[
                    pl.BlockSpec(memory_space=pltpu.MemorySpace.HBM),
                    pl.BlockSpec((SB, TB, PADW), lambda i, j: (i, j, 0)),
                ],
                out_specs=pl.BlockSpec(
                    (TB, D_MODEL, SB), lambda i, j, b=base: (j, 0, b + i)
                ),
                out_shape=jax.ShapeDtypeStruct((T, D_MODEL, S), jnp.float32),
                input_output_aliases={0: 0},
            )(out_t, raw3)
    return out_t.transpose(2, 0, 1)


# speedup vs baseline: 1.2683x; 1.0487x over previous
import functools
import math

import jax
import jax.numpy as jnp
from jax import lax
from jax.experimental import pallas as pl
from jax.experimental.pallas import tpu as pltpu
from jax.experimental.pallas import tpu_sc as plsc

D_MODEL = 64
SCALE = math.sqrt(D_MODEL)
PADW = 128

NC = 2
NS = 16
NW = NC * NS

CHUNK = 512
NBUF = 3

VB = 16384
SB = 512
TB = 8


def _transpose_body(r_ref, o_ref):
    vt = r_ref[...].T
    o_ref[:, :D_MODEL] = vt[: VB // 2]
    o_ref[:, D_MODEL:] = vt[VB // 2 :]


def _gather_body(b_per_w, chunk, table_hbm, idx_hbm, raw_hbm, idx_v, bufs, gsem, wsem):
    wid = lax.axis_index("s") * NC + lax.axis_index("c")
    base = wid * b_per_w
    pltpu.sync_copy(idx_hbm.at[pl.ds(base, b_per_w)], idx_v)

    n = b_per_w // chunk

    def start_gather(g):
        b = g % NBUF
        return pltpu.async_copy(
            table_hbm.at[idx_v.at[pl.ds(g * chunk, chunk)]], bufs[b], gsem.at[b]
        )

    def start_write(g):
        b = g % NBUF
        dst = raw_hbm.at[pl.ds(base + g * chunk, chunk), pl.ds(0, D_MODEL)]
        return pltpu.async_copy(bufs[b], dst, wsem.at[b])

    ghandles = [None] * n
    whandles = [None] * n
    for g in range(n):
        if g >= NBUF:
            whandles[g - NBUF].wait()
        ghandles[g] = start_gather(g)
        if g >= 1:
            ghandles[g - 1].wait()
            whandles[g - 1] = start_write(g - 1)
    ghandles[n - 1].wait()
    whandles[n - 1] = start_write(n - 1)
    for g in range(max(0, n - NBUF), n):
        whandles[g].wait()


def _out_body(r_ref, o_ref):
    r = r_ref[...]
    for t in range(TB):
        rt = r[:, t, :D_MODEL]
        o_ref[t, :, :] = rt.T * SCALE


def _out_body_acc(prev_ref, r_ref, o_ref):
    del prev_ref
    r = r_ref[...]
    for t in range(TB):
        rt = r[:, t, :D_MODEL]
        o_ref[t, :, :] = rt.T * SCALE


def kernel(x, table):
    S, T = x.shape
    B = S * T
    V = table.shape[0]
    b_per_w = B // NW
    assert b_per_w % CHUNK == 0

    n_vb = (V + VB - 1) // VB
    vpad = n_vb * VB
    table_ps = pl.pallas_call(
        _transpose_body,
        grid=(n_vb,),
        in_specs=[pl.BlockSpec((D_MODEL, VB), lambda i: (0, i))],
        out_specs=pl.BlockSpec((VB // 2, 2 * D_MODEL), lambda i: (i, 0)),
        out_shape=jax.ShapeDtypeStruct((vpad // 2, 2 * D_MODEL), jnp.float32),
    )(table.T)
    table_lin = table_ps.reshape(-1).reshape(vpad, D_MODEL)

    h = VB // 2
    xr = (x & ~(VB - 1)) + 2 * (x & (h - 1)) + ((x // h) & 1)

    NSL = 4
    s_sl = S // NSL
    b_sl = s_sl * T
    bw_sl = b_sl // NW
    chunk = 400
    assert bw_sl % chunk == 0

    mesh = plsc.VectorSubcoreMesh(
        core_axis_name="c", subcore_axis_name="s", num_cores=NC
    )
    gather = pl.kernel(
        functools.partial(_gather_body, bw_sl, chunk),
        out_type=jax.ShapeDtypeStruct((b_sl, PADW), jnp.float32),
        mesh=mesh,
        compiler_params=pltpu.CompilerParams(use_tc_tiling_on_sc=False),
        scratch_types=[
            pltpu.VMEM((bw_sl,), jnp.int32),
            [pltpu.VMEM((chunk, D_MODEL), jnp.float32) for _ in range(NBUF)],
            pltpu.SemaphoreType.DMA((NBUF,)),
            pltpu.SemaphoreType.DMA((NBUF,)),
        ],
    )

    out_t = None
    for sl in range(NSL):
        xr_sl = xr[sl * s_sl : (sl + 1) * s_sl].reshape(-1)
        raw = gather(table_lin, xr_sl)
        raw3 = raw.reshape(s_sl, T, PADW)
        base = sl * (s_sl // SB)
        if out_t is None:
            out_t = pl.pallas_call(
                _out_body,
                grid=(s_sl // SB, T // TB),
                in_specs=[
                    pl.BlockSpec((SB, TB, PADW), lambda i, j: (i, j, 0))
                ],
                out_specs=pl.BlockSpec(
                    (TB, D_MODEL, SB), lambda i, j, b=base: (j, 0, b + i)
                ),
                out_shape=jax.ShapeDtypeStruct((T, D_MODEL, S), jnp.float32),
            )(raw3)
        else:
            out_t = pl.pallas_call(
                _out_body_acc,
                grid=(s_sl // SB, T // TB),
                in_specs=[
                    pl.BlockSpec(memory_space=pltpu.MemorySpace.HBM),
                    pl.BlockSpec((SB, TB, PADW), lambda i, j: (i, j, 0)),
                ],
                out_specs=pl.BlockSpec(
                    (TB, D_MODEL, SB), lambda i, j, b=base: (j, 0, b + i)
                ),
                out_shape=jax.ShapeDtypeStruct((T, D_MODEL, S), jnp.float32),
                input_output_aliases={0: 0},
            )(out_t, raw3)
    return out_t.transpose(2, 0, 1)
